# baseline (speedup 1.0000x reference)
import functools

import jax
import jax.numpy as jnp
from jax import lax
from jax.experimental import pallas as pl
from jax.experimental.pallas import tpu as pltpu
from jax.experimental.pallas import tpu_sc as plsc

_NC = 2
_NS = 16
_NL = 16
_NW = _NC * _NS
_KT = 4
_C = 1024
_NB = 2


def _enc_body(xn_ref, xh_ref, wn_ref, wh_ref, on_ref, oh_ref):
    on_ref[...] = jnp.dot(xn_ref[...], wn_ref[...],
                          preferred_element_type=jnp.float32)
    oh_ref[...] = jnp.dot(xh_ref[...], wh_ref[...],
                          preferred_element_type=jnp.float32)


def _encode(x_node_feat, x_he_feat, W_node, W_he):
    N, DF = x_node_feat.shape
    DH = W_node.shape[1]
    BR = 1000
    return pl.pallas_call(
        _enc_body,
        grid=(N // BR,),
        in_specs=[
            pl.BlockSpec((BR, DF), lambda i: (i, 0)),
            pl.BlockSpec((BR, DF), lambda i: (i, 0)),
            pl.BlockSpec((DF, DH), lambda i: (0, 0)),
            pl.BlockSpec((DF, DH), lambda i: (0, 0)),
        ],
        out_specs=[
            pl.BlockSpec((BR, DH), lambda i: (i, 0)),
            pl.BlockSpec((BR, DH), lambda i: (i, 0)),
        ],
        out_shape=[
            jax.ShapeDtypeStruct((N, DH), jnp.float32),
            jax.ShapeDtypeStruct((N, DH), jnp.float32),
        ],
    )(x_node_feat, x_he_feat, W_node, W_he)


_BR = 8


def _fin_body(n_valid, ps_ref, u0_ref, u1_ref, bd_ref, keep_ref, cnt_ref):
    i = pl.program_id(0)
    s = jnp.sum(ps_ref[...], axis=0)
    thr = (jnp.log(-jnp.log(u1_ref[...])) - jnp.log(-jnp.log(u0_ref[...]))
           - bd_ref[...])
    Ccol = keep_ref.shape[1]
    flat = ((i * _BR + lax.broadcasted_iota(jnp.int32, (_BR, Ccol), 0)) * Ccol
            + lax.broadcasted_iota(jnp.int32, (_BR, Ccol), 1))
    keep = jnp.where((flat < n_valid) & (s > thr), 1.0, 0.0)
    keep = keep.astype(jnp.float32)
    keep_ref[...] = keep

    @pl.when(i == 0)
    def _():
        cnt_ref[...] = jnp.zeros((1, 1), jnp.float32)

    cnt_ref[...] += jnp.sum(keep).reshape(1, 1)


def _finalize(ps3, u0, u1, bd_row, n_valid):
    R, Ccol = u0.shape
    return pl.pallas_call(
        functools.partial(_fin_body, n_valid),
        grid=(R // _BR,),
        in_specs=[
            pl.BlockSpec((_NW, _BR, Ccol), lambda i: (0, i, 0)),
            pl.BlockSpec((_BR, Ccol), lambda i: (i, 0)),
            pl.BlockSpec((_BR, Ccol), lambda i: (i, 0)),
            pl.BlockSpec((1, Ccol), lambda i: (0, 0)),
        ],
        out_specs=[
            pl.BlockSpec((_BR, Ccol), lambda i: (i, 0)),
            pl.BlockSpec((1, 1), lambda i: (0, 0)),
        ],
        out_shape=[
            jax.ShapeDtypeStruct((R, Ccol), jnp.float32),
            jax.ShapeDtypeStruct((1, 1), jnp.float32),
        ],
    )(ps3, u0, u1, bd_row)


def _rne_bf16(x):
    b = plsc.bitcast(x, jnp.uint32)
    lsb = (b >> jnp.uint32(16)) & jnp.uint32(1)
    r = (b + jnp.uint32(0x7FFF) + lsb) & jnp.uint32(0xFFFF0000)
    return plsc.bitcast(r, jnp.float32)


def _decode_sc(xt, ht, wd_pad, src_p, dst_p):
    _, _, NNODE = xt.shape
    e_pad = src_p.shape[0]
    nchunk = e_pad // _C
    ngroup = _C // _NL
    mesh = plsc.VectorSubcoreMesh(core_axis_name="c", subcore_axis_name="s")

    @functools.partial(
        pl.kernel,
        mesh=mesh,
        out_type=jax.ShapeDtypeStruct((_NW, e_pad), jnp.float32),
        scratch_types=[
            pltpu.VMEM((_KT, NNODE), jnp.float32),
            pltpu.VMEM((_KT, NNODE), jnp.float32),
            pltpu.VMEM((_NB, _C), jnp.int32),
            pltpu.VMEM((_NB, _C), jnp.int32),
            pltpu.VMEM((_NB, _C), jnp.float32),
            pltpu.VMEM((_NW * _KT + _NL,), jnp.float32),
        ] + [pltpu.SemaphoreType.DMA] * 6,
        compiler_params=pltpu.CompilerParams(needs_layout_passes=False,
                                             use_tc_tiling_on_sc=False),
    )
    def k(xt_hbm, ht_hbm, wd_hbm, src_hbm, dst_hbm, psum_hbm,
          ta, tb, srcr, dstr, psr, wdv, sS0, sD0, sS1, sD1, sP0, sP1):
        sid = lax.axis_index("s")
        cid = lax.axis_index("c")
        wid = sid * _NC + cid
        iota16 = lax.iota(jnp.int32, _NL)
        ssem = (sS0, sS1)
        dsem = (sD0, sD1)
        psem = (sP0, sP1)

        pltpu.sync_copy(xt_hbm.at[wid], ta)
        pltpu.sync_copy(ht_hbm.at[wid], tb)
        pltpu.sync_copy(wd_hbm, wdv)
        wsl = wdv[pl.ds(wid * _KT, _NL)]
        wks = [wsl[j] for j in range(_KT)]
        kvecs = [jnp.zeros((_NL,), jnp.int32) + j for j in range(_KT)]

        def stage(ci, b):
            pltpu.async_copy(src_hbm.at[pl.ds(ci * _C, _C)], srcr.at[b],
                             ssem[b])
            pltpu.async_copy(dst_hbm.at[pl.ds(ci * _C, _C)], dstr.at[b],
                             dsem[b])

        def wait_stage(ci, b):
            pltpu.make_async_copy(src_hbm.at[pl.ds(ci * _C, _C)], srcr.at[b],
                                  ssem[b]).wait()
            pltpu.make_async_copy(dst_hbm.at[pl.ds(ci * _C, _C)], dstr.at[b],
                                  dsem[b]).wait()

        def ps_dst(ci):
            return psum_hbm.at[wid].at[pl.ds(ci * _C, _C)]

        def store_ps(ci, b):
            pltpu.async_copy(psr.at[b], ps_dst(ci), psem[b])

        def wait_ps(ci, b):
            pltpu.make_async_copy(psr.at[b], ps_dst(ci), psem[b]).wait()

        def compute(ci, b):
            def group_body(g4, _):
                for u in range(4):
                    off = (g4 * 4 + u) * _NL
                    srcv = srcr[b, pl.ds(off, _NL)]
                    dstv = dstr[b, pl.ds(off, _NL)]
                    acc = jnp.zeros((_NL,), jnp.float32)
                    a1 = jnp.zeros((_NL,), jnp.float32)
                    for j in range(_KT):
                        ga = plsc.load_gather(ta, [kvecs[j], srcv])
                        gb = plsc.load_gather(tb, [kvecs[j], dstv])
                        p = _rne_bf16(ga * gb)
                        if j % 2 == 0:
                            acc = acc + p * wks[j]
                        else:
                            a1 = a1 + p * wks[j]
                    psr[b, pl.ds(off, _NL)] = acc + a1
                return 0

            lax.fori_loop(0, ngroup // 4, group_body, 0)

        stage(0, 0)
        stage(1, 1)

        def pair_body(jp, _):
            for b in range(_NB):
                ci = jp * _NB + b
                wait_stage(ci, b)

                @pl.when(ci >= _NB)
                def _():
                    wait_ps(ci - _NB, b)

                compute(ci, b)
                store_ps(ci, b)

                @pl.when(ci + _NB < nchunk)
                def _():
                    stage(ci + _NB, b)
            return 0

        lax.fori_loop(0, nchunk // _NB, pair_body, 0)
        wait_ps(nchunk - 2, 0)
        wait_ps(nchunk - 1, 1)

    return k(xt, ht, wd_pad, src_p, dst_p)


def kernel(x_node_feat, x_he_feat, W_node, W_he, W_dec, b_dec, edge_index,
           num_ori_edge, gumbel_u):
    n_ori = gumbel_u.shape[0]
    n_edges = edge_index.shape[1]
    DH = W_node.shape[1]
    blk = _C * _NB
    e_pad = ((n_ori + blk - 1) // blk) * blk

    wdb = (W_dec[:, 1].astype(jnp.bfloat16).astype(jnp.float32)
           - W_dec[:, 0].astype(jnp.bfloat16).astype(jnp.float32))
    wd_pad = jnp.pad(wdb, (0, _NL))
    bd = b_dec[1] - b_dec[0]
    gcol = 128
    bd_row = jnp.full((1, gcol), bd, jnp.float32)

    zero_dep = jnp.asarray(num_ori_edge, dtype=edge_index.dtype) - n_ori
    src_p = jnp.pad(edge_index[0, :n_ori] + zero_dep,
                    (0, e_pad - n_ori)).astype(jnp.int32)
    dst_p = jnp.pad(edge_index[1, :n_ori] + zero_dep,
                    (0, e_pad - n_ori)).astype(jnp.int32)

    gup = jnp.pad(gumbel_u, ((0, e_pad - n_ori), (0, 0)), constant_values=0.5)
    R = e_pad // gcol
    u0 = gup[:, 0].reshape(R, gcol)
    u1 = gup[:, 1].reshape(R, gcol)

    xn, xh = _encode(x_node_feat, x_he_feat, W_node, W_he)
    N = xn.shape[0]
    xt = xn.T.reshape(_NW, _KT, N)
    ht = xh.T.reshape(_NW, _KT, N)

    psum = _decode_sc(xt, ht, wd_pad, src_p, dst_p)

    ps3 = psum.reshape(_NW, R, gcol)
    keep2d, cnt = _finalize(ps3, u0, u1, bd_row, n_ori)

    keep = keep2d.reshape(e_pad)[:n_ori]
    deg = 1.0 - cnt[0, 0] / jnp.float32(n_ori)
    full = jnp.concatenate(
        [keep, jnp.ones((n_edges - n_ori,), jnp.float32)], axis=0)
    return (full, deg)


# speedup vs baseline: 1.5310x; 1.5310x over previous
import functools

import jax
import jax.numpy as jnp
from jax import lax
from jax.experimental import pallas as pl
from jax.experimental.pallas import tpu as pltpu
from jax.experimental.pallas import tpu_sc as plsc

_NC = 2
_NS = 16
_NL = 16
_NW = _NC * _NS
_C = 64
_NB = 4


def _enc_body(xn_ref, xh_ref, wn_ref, wh_ref, on_ref, oh_ref):
    on_ref[...] = jnp.dot(xn_ref[...], wn_ref[...],
                          preferred_element_type=jnp.float32)
    oh_ref[...] = jnp.dot(xh_ref[...], wh_ref[...],
                          preferred_element_type=jnp.float32)


def _encode(x_node_feat, x_he_feat, W_node, W_he):
    N, DF = x_node_feat.shape
    DH = W_node.shape[1]
    BR = 1000
    return pl.pallas_call(
        _enc_body,
        grid=(N // BR,),
        in_specs=[
            pl.BlockSpec((BR, DF), lambda i: (i, 0)),
            pl.BlockSpec((BR, DF), lambda i: (i, 0)),
            pl.BlockSpec((DF, DH), lambda i: (0, 0)),
            pl.BlockSpec((DF, DH), lambda i: (0, 0)),
        ],
        out_specs=[
            pl.BlockSpec((BR, DH), lambda i: (i, 0)),
            pl.BlockSpec((BR, DH), lambda i: (i, 0)),
        ],
        out_shape=[
            jax.ShapeDtypeStruct((N, DH), jnp.float32),
            jax.ShapeDtypeStruct((N, DH), jnp.float32),
        ],
    )(x_node_feat, x_he_feat, W_node, W_he)


def _gum_body(n_valid, u0_ref, u1_ref, bd_ref, thr_ref):
    t = (jnp.log(-jnp.log(u1_ref[...])) - jnp.log(-jnp.log(u0_ref[...]))
         - bd_ref[...])
    R, Ccol = thr_ref.shape
    flat = (lax.broadcasted_iota(jnp.int32, (R, Ccol), 0) * Ccol
            + lax.broadcasted_iota(jnp.int32, (R, Ccol), 1))
    thr_ref[...] = jnp.where(flat < n_valid, t, jnp.inf)


def _gumbel_thr(u0, u1, bd_row, n_valid):
    R, Ccol = u0.shape
    return pl.pallas_call(
        functools.partial(_gum_body, n_valid),
        out_shape=jax.ShapeDtypeStruct((R, Ccol), jnp.float32),
    )(u0, u1, bd_row)


def _rne_bf16(x):
    b = plsc.bitcast(x, jnp.uint32)
    lsb = (b >> jnp.uint32(16)) & jnp.uint32(1)
    r = (b + jnp.uint32(0x7FFF) + lsb) & jnp.uint32(0xFFFF0000)
    return plsc.bitcast(r, jnp.float32)


def _decode_sc(xn, xhd, wdb, src_p, dst_p, thr_p):
    e_pad = thr_p.shape[0]
    NNODE, DH = xn.shape
    per_w = e_pad // _NW
    nchunk = per_w // _C
    ngroup = _C // _NL
    mesh = plsc.VectorSubcoreMesh(core_axis_name="c", subcore_axis_name="s")

    @functools.partial(
        pl.kernel,
        mesh=mesh,
        out_type=[
            jax.ShapeDtypeStruct((e_pad,), jnp.float32),
            jax.ShapeDtypeStruct((_NW, _NL), jnp.float32),
        ],
        scratch_types=[
            pltpu.VMEM((nchunk, _C), jnp.int32),
            pltpu.VMEM((nchunk, _C), jnp.int32),
            pltpu.VMEM((per_w,), jnp.float32),
            pltpu.VMEM((per_w,), jnp.float32),
            pltpu.VMEM((_NB, _C, DH), jnp.float32),
            pltpu.VMEM((_NB, _C, DH), jnp.float32),
            pltpu.VMEM((DH,), jnp.float32),
            pltpu.VMEM((_NL,), jnp.float32),
        ] + [pltpu.SemaphoreType.DMA] * (2 * _NB),
        compiler_params=pltpu.CompilerParams(needs_layout_passes=False),
    )
    def k(xn_hbm, xh_hbm, wd_hbm, src_hbm, dst_hbm, thr_hbm, keep_hbm, cnt_hbm,
          srcv, dstv, thrv, keepv, av, bv, wdv, cntv, *allsems):
        wid = lax.axis_index("s") * _NC + lax.axis_index("c")
        base_w = wid * per_w
        iota16 = lax.iota(jnp.int32, _NL)
        sems = tuple((allsems[2 * i], allsems[2 * i + 1]) for i in range(_NB))
        c_offs = [(iota16 + j) & (_NL - 1) for j in range(_NL)]

        pltpu.sync_copy(wd_hbm, wdv)
        pltpu.sync_copy(src_hbm.at[wid], srcv)
        pltpu.sync_copy(dst_hbm.at[wid], dstv)
        pltpu.sync_copy(thr_hbm.at[pl.ds(base_w, per_w)], thrv)

        def start(ci, slot):
            pltpu.async_copy(xn_hbm.at[srcv.at[ci]],
                             av.at[slot], sems[slot][0])
            pltpu.async_copy(xh_hbm.at[dstv.at[ci]],
                             bv.at[slot], sems[slot][1])

        def wait(ci, slot):
            pltpu.make_async_copy(xn_hbm.at[srcv.at[ci]],
                                  av.at[slot], sems[slot][0]).wait()
            pltpu.make_async_copy(xh_hbm.at[dstv.at[ci]],
                                  bv.at[slot], sems[slot][1]).wait()

        def compute(ci, slot):
            a2d = av.at[slot]
            b2d = bv.at[slot]

            def group_body(g, _):
                rows = g * _NL + iota16

                accs = [jnp.zeros((_NL,), jnp.float32) for _ in range(8)]
                for k in range(DH):
                    s, j = divmod(k, _NL)
                    c = c_offs[j] + (s * _NL)
                    p = _rne_bf16(plsc.load_gather(a2d, [rows, c])
                                  * plsc.load_gather(b2d, [rows, c]))
                    w = plsc.load_gather(wdv, [c])
                    accs[k % 8] = accs[k % 8] + p * w
                acc = (((accs[0] + accs[1]) + (accs[2] + accs[3]))
                       + ((accs[4] + accs[5]) + (accs[6] + accs[7])))
                off = ci * _C + g * _NL
                thrg = thrv[pl.ds(off, _NL)]
                keep = jnp.where(acc > thrg, 1.0, 0.0).astype(jnp.float32)
                keepv[pl.ds(off, _NL)] = keep
                return 0

            lax.fori_loop(0, ngroup, group_body, 0)

        for b in range(_NB):
            start(b, b)

        def outer_body(po, _):
            for b in range(_NB):
                ci = po * _NB + b
                wait(ci, b)
                compute(ci, b)

                @pl.when(ci + _NB < nchunk)
                def _():
                    start(ci + _NB, b)
            return 0

        lax.fori_loop(0, nchunk // _NB, outer_body, 0)

        def cnt_body(i, acc):
            return acc + keepv[pl.ds(i * _NL, _NL)]

        cnt = lax.fori_loop(0, per_w // _NL, cnt_body,
                            jnp.zeros((_NL,), jnp.float32))
        cntv[...] = cnt
        pltpu.sync_copy(keepv, keep_hbm.at[pl.ds(base_w, per_w)])
        pltpu.sync_copy(cntv, cnt_hbm.at[wid])

    return k(xn, xhd, wdb, src_p, dst_p, thr_p)


def kernel(x_node_feat, x_he_feat, W_node, W_he, W_dec, b_dec, edge_index,
           num_ori_edge, gumbel_u):
    n_ori = gumbel_u.shape[0]
    n_edges = edge_index.shape[1]
    DH = W_node.shape[1]
    blk = _NW * _C * _NB
    e_pad = ((n_ori + blk - 1) // blk) * blk

    wdb = (W_dec[:, 1].astype(jnp.bfloat16).astype(jnp.float32)
           - W_dec[:, 0].astype(jnp.bfloat16).astype(jnp.float32))
    bd = b_dec[1] - b_dec[0]
    gcol = 128
    bd_row = jnp.full((1, gcol), bd, jnp.float32)

    zero_dep = jnp.asarray(num_ori_edge, dtype=edge_index.dtype) - n_ori
    src_p = jnp.pad(edge_index[0, :n_ori] + zero_dep,
                    (0, e_pad - n_ori)).astype(jnp.int32)
    dst_p = jnp.pad(edge_index[1, :n_ori] + zero_dep,
                    (0, e_pad - n_ori)).astype(jnp.int32)

    gup = jnp.pad(gumbel_u, ((0, e_pad - n_ori), (0, 0)), constant_values=0.5)
    R = e_pad // gcol
    u0 = gup[:, 0].reshape(R, gcol)
    u1 = gup[:, 1].reshape(R, gcol)

    xn, xhd = _encode(x_node_feat, x_he_feat, W_node, W_he)
    thr_p = _gumbel_thr(u0, u1, bd_row, n_ori).reshape(e_pad)

    nchunk = e_pad // (_NW * _C)
    src_3d = src_p.reshape(_NW, nchunk, _C)
    dst_3d = dst_p.reshape(_NW, nchunk, _C)
    keep_p, counts = _decode_sc(xn, xhd, wdb, src_3d, dst_3d, thr_p)

    keep = keep_p[:n_ori]
    deg = 1.0 - jnp.sum(counts) / jnp.float32(n_ori)
    full = jnp.concatenate(
        [keep, jnp.ones((n_edges - n_ori,), jnp.float32)], axis=0)
    return (full, deg)
